# baseline (speedup 1.0000x reference)
import functools

import jax
import jax.numpy as jnp
from jax import lax
from jax.experimental import pallas as pl
from jax.experimental.pallas import tpu as pltpu
from jax.experimental.pallas import tpu_sc as plsc

B = 16384
DA = 10
DB = 50
DP = 64
G = 4
DQ = G * DP
NQ = 4 ** G
NC = 2
NS = 16
NW = NC * NS
QPW = (B // G) // NW

_MESH = plsc.VectorSubcoreMesh(core_axis_name="c", subcore_axis_name="s")


@functools.partial(
    pl.kernel,
    mesh=_MESH,
    out_type=jax.ShapeDtypeStruct((B // G, DQ), jnp.float32),
    scratch_types=[
        pltpu.VMEM((QPW,), jnp.int32),
        pltpu.VMEM((QPW, DQ), jnp.float32),
        pltpu.SemaphoreType.DMA,
    ],
)
def _sc_gather(qc_hbm, tab_hbm, out_hbm, qc_v, rows_v, sem):
    wid = lax.axis_index("s") * NC + lax.axis_index("c")
    pltpu.sync_copy(qc_hbm.at[wid], qc_v)
    pltpu.async_copy(tab_hbm.at[qc_v], rows_v, sem).wait()
    pltpu.sync_copy(rows_v, out_hbm.at[pl.ds(wid * QPW, QPW)])


def kernel(index, a, b):
    idx = index.astype(jnp.int32).reshape(B // G, G)
    qc = ((idx[:, 0] * 4 + idx[:, 1]) * 4 + idx[:, 2]) * 4 + idx[:, 3]
    qc = qc.reshape(NW, QPW)
    tab = jnp.concatenate(
        [a.reshape(4, DA), b.reshape(4, DB),
         jnp.zeros((4, DP - DA - DB), jnp.float32)], axis=1)
    q = jnp.arange(NQ, dtype=jnp.int32)
    digits = jnp.stack(
        [(q >> (2 * (G - 1 - c))) & 3 for c in range(G)], axis=1)
    tab_quad = tab[digits].reshape(NQ, DQ)
    out = _sc_gather(qc, tab_quad).reshape(B // G, G, DP)
    return (out[:, :, :DA].reshape(B, 2, 5),
            out[:, :, DA:DA + DB].reshape(B, 2, 5, 5))


# speedup vs baseline: 4.0093x; 4.0093x over previous
import functools

import jax
import jax.numpy as jnp
from jax import lax
from jax.experimental import pallas as pl
from jax.experimental.pallas import tpu as pltpu
from jax.experimental.pallas import tpu_sc as plsc

B = 16384
DA = 10
DB = 50
DP = 64
G = 4
DQ = G * DP
NQ = 4 ** G
NC = 2
NS = 16
NW = NC * NS
QPW = (B // G) // NW

_MESH = plsc.VectorSubcoreMesh(core_axis_name="c", subcore_axis_name="s")


@functools.partial(
    pl.kernel,
    mesh=_MESH,
    out_type=jax.ShapeDtypeStruct((B // G, DQ), jnp.float32),
    scratch_types=[
        pltpu.VMEM((QPW,), jnp.int32),
        pltpu.VMEM((QPW, DQ), jnp.float32),
        pltpu.SemaphoreType.DMA,
    ],
)
def _sc_gather(qc_hbm, tab_hbm, out_hbm, qc_v, rows_v, sem):
    wid = lax.axis_index("s") * NC + lax.axis_index("c")
    pltpu.sync_copy(qc_hbm.at[wid], qc_v)
    pltpu.async_copy(tab_hbm.at[qc_v], rows_v, sem).wait()
    pltpu.sync_copy(rows_v, out_hbm.at[pl.ds(wid * QPW, QPW)])


def kernel(index, a, b):
    idx = index.astype(jnp.int32).reshape(B // G, G)
    qc = ((idx[:, 0] * 4 + idx[:, 1]) * 4 + idx[:, 2]) * 4 + idx[:, 3]
    qc = qc.reshape(NW, QPW)
    tab = jnp.concatenate(
        [a.reshape(4, DA), b.reshape(4, DB),
         jnp.zeros((4, DP - DA - DB), jnp.float32)], axis=1)
    q = jnp.arange(NQ, dtype=jnp.int32)
    digits = jnp.stack(
        [(q >> (2 * (G - 1 - c))) & 3 for c in range(G)], axis=1)
    tab_quad = tab[digits].reshape(NQ, DQ)
    out = _sc_gather(qc, tab_quad).reshape(B, DP)
    return (out[:, :DA].reshape(B, 2, 5),
            out[:, DA:DA + DB].reshape(B, 2, 5, 5))
